# baseline (speedup 1.0000x reference)
import functools

import jax
import jax.numpy as jnp
from jax import lax
from jax.experimental import pallas as pl
from jax.experimental.pallas import tpu as pltpu
from jax.experimental.pallas import tpu_sc as plsc

_B, _I, _N, _M = 8, 256, 4096, 16384
_L = 16
_NC, _NS = 2, 16
_NW = _NC * _NS
_ROWS = _I // _NW
_CHUNK = 2048
_NCHUNK = _M // _CHUNK
_NG = _B * _NCHUNK


def _gather_features(inp, idx):
    mesh = plsc.VectorSubcoreMesh(core_axis_name="c", subcore_axis_name="s")

    @functools.partial(
        pl.kernel,
        mesh=mesh,
        out_type=jax.ShapeDtypeStruct((_B, _I, _M), jnp.float32),
        compiler_params=pltpu.CompilerParams(needs_layout_passes=False),
        scratch_types=[
            pltpu.VMEM((_ROWS, _N), jnp.float32),
            pltpu.VMEM((_M,), jnp.int32),
            pltpu.VMEM((_ROWS, _CHUNK), jnp.float32),
            pltpu.VMEM((_ROWS, _CHUNK), jnp.float32),
            pltpu.VMEM((_ROWS, _CHUNK), jnp.float32),
            pltpu.VMEM((_ROWS, _CHUNK), jnp.float32),
            pltpu.SemaphoreType.DMA,
            pltpu.SemaphoreType.DMA,
            pltpu.SemaphoreType.DMA,
            pltpu.SemaphoreType.DMA,
        ],
    )
    def k(inp_hbm, idx_hbm, out_hbm, rows_v, idx_v, out0, out1, out2, out3,
          sem0, sem1, sem2, sem3):
        wid = lax.axis_index("s") * _NC + lax.axis_index("c")
        row_base = wid * _ROWS
        bufs = (out0, out1, out2, out3)
        sems = (sem0, sem1, sem2, sem3)

        def out_slice(g):
            b, c = g >> 3, g & 7
            return out_hbm.at[
                b, pl.ds(row_base, _ROWS), pl.ds(c * _CHUNK, _CHUNK)]

        def do_chunk(g, buf, sem):
            m0 = (g & 7) * _CHUNK

            @functools.partial(plsc.parallel_loop, 0, _CHUNK // _L, unroll=8)
            def per_vec(j):
                iv = idx_v[pl.ds(m0 + j * _L, _L)]
                for r in range(_ROWS):
                    rv = jnp.full((_L,), r, jnp.int32)
                    buf[r, pl.ds(j * _L, _L)] = plsc.load_gather(
                        rows_v, [rv, iv])

            pltpu.async_copy(buf, out_slice(g), sem)

        def per_quad(q, _):
            g = q * 4
            b = g >> 3

            @pl.when((g & 7) == 0)
            def _():
                pltpu.sync_copy(inp_hbm.at[b, pl.ds(row_base, _ROWS)], rows_v)
                pltpu.sync_copy(idx_hbm.at[b], idx_v)

            for s in range(4):
                @pl.when(q >= 1)
                def _(s=s):
                    pltpu.make_async_copy(
                        bufs[s], out_slice(g + s - 4), sems[s]).wait()

                do_chunk(g + s, bufs[s], sems[s])
            return 0

        lax.fori_loop(0, _NG // 4, per_quad, 0)
        for s in range(4):
            pltpu.make_async_copy(
                bufs[s], out_slice(_NG - 4 + s), sems[s]).wait()

    return k(inp, idx)


def kernel(input, points, support_points, indices):
    idx = indices[:, :, 0]
    features = _gather_features(input, idx)
    return features, support_points, indices


# speedup vs baseline: 1.0373x; 1.0373x over previous
import functools

import jax
import jax.numpy as jnp
from jax import lax
from jax.experimental import pallas as pl
from jax.experimental.pallas import tpu as pltpu
from jax.experimental.pallas import tpu_sc as plsc

_B, _I, _N, _M = 8, 256, 4096, 16384
_L = 16
_NC, _NS = 2, 16
_NW = _NC * _NS
_ROWS = _I // _NW
_CHUNK = 2048
_NCHUNK = _M // _CHUNK
_NG = _B * _NCHUNK


def _gather_features(inp, idx):
    mesh = plsc.VectorSubcoreMesh(core_axis_name="c", subcore_axis_name="s")

    @functools.partial(
        pl.kernel,
        mesh=mesh,
        out_type=jax.ShapeDtypeStruct((_B, _I, _M), jnp.float32),
        compiler_params=pltpu.CompilerParams(needs_layout_passes=False),
        scratch_types=[
            pltpu.VMEM((_ROWS, _N), jnp.float32),
            pltpu.VMEM((_M,), jnp.int32),
            pltpu.VMEM((_ROWS, _CHUNK), jnp.float32),
            pltpu.VMEM((_ROWS, _CHUNK), jnp.float32),
            pltpu.SemaphoreType.DMA,
            pltpu.SemaphoreType.DMA,
        ],
    )
    def k(inp_hbm, idx_hbm, out_hbm, rows_v, idx_v, out0, out1, sem0, sem1):
        wid = lax.axis_index("s") * _NC + lax.axis_index("c")
        row_base = wid * _ROWS
        bufs = (out0, out1)
        sems = (sem0, sem1)

        def out_slice(g):
            b, c = g >> 3, g & 7
            return out_hbm.at[
                b, pl.ds(row_base, _ROWS), pl.ds(c * _CHUNK, _CHUNK)]

        def do_chunk(g, buf, sem):
            m0 = (g & 7) * _CHUNK

            @functools.partial(plsc.parallel_loop, 0, _CHUNK // _L, unroll=4)
            def per_vec(j):
                iv = idx_v[pl.ds(m0 + j * _L, _L)]
                for r in range(_ROWS):
                    rv = jnp.full((_L,), r, jnp.int32)
                    buf[r, pl.ds(j * _L, _L)] = plsc.load_gather(
                        rows_v, [rv, iv])

            pltpu.async_copy(buf, out_slice(g), sem)

        def per_pair(g2, _):
            g = g2 * 2
            b = g >> 3

            @pl.when((g & 7) == 0)
            def _():
                pltpu.sync_copy(inp_hbm.at[b, pl.ds(row_base, _ROWS)], rows_v)
                pltpu.sync_copy(idx_hbm.at[b], idx_v)

            @pl.when(g2 >= 1)
            def _():
                pltpu.make_async_copy(out0, out_slice(g - 2), sem0).wait()

            do_chunk(g, out0, sem0)

            @pl.when(g2 >= 1)
            def _():
                pltpu.make_async_copy(out1, out_slice(g - 1), sem1).wait()

            do_chunk(g + 1, out1, sem1)
            return 0

        lax.fori_loop(0, _NG // 2, per_pair, 0)
        pltpu.make_async_copy(out0, out_slice(_NG - 2), sem0).wait()
        pltpu.make_async_copy(out1, out_slice(_NG - 1), sem1).wait()

    return k(inp, idx)


def kernel(input, points, support_points, indices):
    idx = indices[:, :, 0]
    features = _gather_features(input, idx)
    return features, support_points, indices
